# Initial kernel scaffold; baseline (speedup 1.0000x reference)
import jax
import jax.numpy as jnp
from jax.experimental import pallas as pl


def kernel(edge_weight, edge_index):
    raise NotImplementedError("write your pallas kernel here")


# speedup vs baseline: 250.1841x; 250.1841x over previous
import functools

import jax
import jax.numpy as jnp
from jax import lax
from jax.experimental import pallas as pl
from jax.experimental.pallas import tpu as pltpu
from jax.experimental.pallas import tpu_sc as plsc

N_NODES_ = 100000
N_EDGES_ = 6400000

NPAD = 100096
PT_NODES = NPAD // 16

CHUNK = 8000
CHUNKS_A = (N_EDGES_ // 16) // CHUNK
PER_W = N_EDGES_ // 32
CHUNKS_B = PER_W // CHUNK

_MESH = plsc.VectorSubcoreMesh(core_axis_name="c", subcore_axis_name="s")


def _rsqrt16(x):
    i = lax.bitcast_convert_type(x, jnp.int32)
    i = jnp.int32(0x5F3759DF) - lax.shift_right_logical(i, 1)
    y = lax.bitcast_convert_type(i, jnp.float32)
    for _ in range(3):
        y = y * (1.5 - 0.5 * x * y * y)
    return y


@functools.partial(
    pl.kernel,
    mesh=_MESH,
    out_type=(
        jax.ShapeDtypeStruct((NPAD,), jnp.float32),
        jax.ShapeDtypeStruct((NPAD,), jnp.float32),
    ),
    scratch_types=[
        pltpu.VMEM_SHARED((NPAD,), jnp.float32),
        pltpu.VMEM((CHUNK,), jnp.int32),
        pltpu.VMEM((CHUNK,), jnp.float32),
        pltpu.VMEM((PT_NODES,), jnp.float32),
    ],
)
def _norms(ew_hbm, src_hbm, dst_hbm, rs_out_hbm, rs_in_hbm,
           acc, idx_v, val_v, node_v):
    cid = lax.axis_index("c")
    sid = lax.axis_index("s")
    nbase = sid * PT_NODES

    def _zero(i, carry):
        node_v[pl.ds(i * 16, 16)] = jnp.zeros((16,), jnp.float32)
        return carry
    lax.fori_loop(0, PT_NODES // 16, _zero, 0)
    pltpu.sync_copy(node_v, acc.at[pl.ds(nbase, PT_NODES)])
    plsc.subcore_barrier()

    def _scat(c, carry):
        base = (sid * CHUNKS_A + c) * CHUNK

        @pl.when(cid == 0)
        def _():
            pltpu.sync_copy(src_hbm.at[pl.ds(base, CHUNK)], idx_v)

        @pl.when(cid == 1)
        def _():
            pltpu.sync_copy(dst_hbm.at[pl.ds(base, CHUNK)], idx_v)

        pltpu.sync_copy(ew_hbm.at[pl.ds(base, CHUNK)], val_v)
        pltpu.sync_copy(val_v, acc.at[idx_v], add=True)
        return carry
    lax.fori_loop(0, CHUNKS_A, _scat, 0)
    plsc.subcore_barrier()

    pltpu.sync_copy(acc.at[pl.ds(nbase, PT_NODES)], node_v)

    def _rs(i, carry):
        s = pl.ds(i * 16, 16)
        node_v[s] = _rsqrt16(node_v[s])
        return carry
    lax.fori_loop(0, PT_NODES // 16, _rs, 0)

    @pl.when(cid == 0)
    def _():
        pltpu.sync_copy(node_v, rs_out_hbm.at[pl.ds(nbase, PT_NODES)])

    @pl.when(cid == 1)
    def _():
        pltpu.sync_copy(node_v, rs_in_hbm.at[pl.ds(nbase, PT_NODES)])


@functools.partial(
    pl.kernel,
    mesh=_MESH,
    out_type=jax.ShapeDtypeStruct((N_EDGES_,), jnp.float32),
    scratch_types=[
        pltpu.VMEM_SHARED((NPAD,), jnp.float32),
        pltpu.VMEM_SHARED((NPAD,), jnp.float32),
        pltpu.VMEM((CHUNK,), jnp.int32),
        pltpu.VMEM((CHUNK,), jnp.int32),
        pltpu.VMEM((CHUNK,), jnp.float32),
        pltpu.VMEM((CHUNK,), jnp.float32),
        pltpu.VMEM((CHUNK,), jnp.float32),
        pltpu.VMEM((CHUNK,), jnp.float32),
    ],
)
def _apply(ew_hbm, src_hbm, dst_hbm, rs_out_hbm, rs_in_hbm, out_hbm,
           rso_s, rsi_s, src_v, dst_v, ew_v, a_v, b_v, o_v):
    cid = lax.axis_index("c")
    sid = lax.axis_index("s")
    wid = sid * 2 + cid
    nbase = sid * PT_NODES
    ns = pl.ds(nbase, PT_NODES)

    pltpu.sync_copy(rs_out_hbm.at[ns], a_v.at[pl.ds(0, PT_NODES)])
    pltpu.sync_copy(a_v.at[pl.ds(0, PT_NODES)], rso_s.at[ns])
    pltpu.sync_copy(rs_in_hbm.at[ns], b_v.at[pl.ds(0, PT_NODES)])
    pltpu.sync_copy(b_v.at[pl.ds(0, PT_NODES)], rsi_s.at[ns])
    plsc.subcore_barrier()

    def _chunk(c, carry):
        base = wid * PER_W + c * CHUNK
        pltpu.sync_copy(src_hbm.at[pl.ds(base, CHUNK)], src_v)
        pltpu.sync_copy(dst_hbm.at[pl.ds(base, CHUNK)], dst_v)
        pltpu.sync_copy(ew_hbm.at[pl.ds(base, CHUNK)], ew_v)
        pltpu.sync_copy(rso_s.at[src_v], a_v)
        pltpu.sync_copy(rsi_s.at[dst_v], b_v)

        def _mul(i, carry2):
            s = pl.ds(i * 16, 16)
            o_v[s] = a_v[s] * b_v[s] * ew_v[s]
            return carry2
        lax.fori_loop(0, CHUNK // 16, _mul, 0)
        pltpu.sync_copy(o_v, out_hbm.at[pl.ds(base, CHUNK)])
        return carry
    lax.fori_loop(0, CHUNKS_B, _chunk, 0)


@jax.jit
def kernel(edge_weight, edge_index):
    ew = edge_weight.astype(jnp.float32)
    idx = edge_index.astype(jnp.int32)
    src = idx[0]
    dst = idx[1]
    rs_out, rs_in = _norms(ew, src, dst)
    return _apply(ew, src, dst, rs_out, rs_in)


# speedup vs baseline: 286.7768x; 1.1463x over previous
import functools

import jax
import jax.numpy as jnp
from jax import lax
from jax.experimental import pallas as pl
from jax.experimental.pallas import tpu as pltpu
from jax.experimental.pallas import tpu_sc as plsc

N_NODES_ = 100000
N_EDGES_ = 6400000

NPAD = 100096
PT_NODES = NPAD // 16

CHUNK_A = 10000
CHUNKS_A = (N_EDGES_ // 16) // CHUNK_A
CHUNK_B = 4000
PER_W = N_EDGES_ // 32
CHUNKS_B = PER_W // CHUNK_B
MUL_UNROLL = 10

_MESH = plsc.VectorSubcoreMesh(core_axis_name="c", subcore_axis_name="s")


def _rsqrt16(x):
    i = lax.bitcast_convert_type(x, jnp.int32)
    i = jnp.int32(0x5F3759DF) - lax.shift_right_logical(i, 1)
    y = lax.bitcast_convert_type(i, jnp.float32)
    for _ in range(3):
        y = y * (1.5 - 0.5 * x * y * y)
    return y


@functools.partial(
    pl.kernel,
    mesh=_MESH,
    out_type=jax.ShapeDtypeStruct((2 * NPAD,), jnp.float32),
    scratch_types=[
        pltpu.VMEM_SHARED((NPAD,), jnp.float32),
        pltpu.VMEM((CHUNK_A,), jnp.int32),
        pltpu.VMEM((CHUNK_A,), jnp.float32),
        pltpu.VMEM((CHUNK_A,), jnp.int32),
        pltpu.VMEM((CHUNK_A,), jnp.float32),
        pltpu.VMEM((PT_NODES,), jnp.float32),
        pltpu.SemaphoreType.DMA,
        pltpu.SemaphoreType.DMA,
        pltpu.SemaphoreType.DMA,
        pltpu.SemaphoreType.DMA,
    ],
)
def _norms(ew_hbm, ei_hbm, rs_hbm,
           acc, idx0, val0, idx1, val1, node_v, si0, sv0, si1, sv1):
    cid = lax.axis_index("c")
    sid = lax.axis_index("s")
    nbase = sid * PT_NODES

    def _zero(i, carry):
        node_v[pl.ds(i * 16, 16)] = jnp.zeros((16,), jnp.float32)
        return carry
    lax.fori_loop(0, PT_NODES // 16, _zero, 0)
    pltpu.sync_copy(node_v, acc.at[pl.ds(nbase, PT_NODES)])
    plsc.subcore_barrier()

    def _scat(k, carry):
        b0 = (sid * CHUNKS_A + 2 * k) * CHUNK_A
        b1 = b0 + CHUNK_A
        row = cid * N_EDGES_
        ci0 = pltpu.async_copy(ei_hbm.at[pl.ds(row + b0, CHUNK_A)], idx0, si0)
        cv0 = pltpu.async_copy(ew_hbm.at[pl.ds(b0, CHUNK_A)], val0, sv0)
        ci1 = pltpu.async_copy(ei_hbm.at[pl.ds(row + b1, CHUNK_A)], idx1, si1)
        cv1 = pltpu.async_copy(ew_hbm.at[pl.ds(b1, CHUNK_A)], val1, sv1)
        ci0.wait()
        cv0.wait()
        pltpu.sync_copy(val0, acc.at[idx0], add=True)
        ci1.wait()
        cv1.wait()
        pltpu.sync_copy(val1, acc.at[idx1], add=True)
        return carry
    lax.fori_loop(0, CHUNKS_A // 2, _scat, 0)
    plsc.subcore_barrier()

    pltpu.sync_copy(acc.at[pl.ds(nbase, PT_NODES)], node_v)

    def _rs(i, carry):
        s = pl.ds(i * 16, 16)
        node_v[s] = _rsqrt16(node_v[s])
        return carry
    lax.fori_loop(0, PT_NODES // 16, _rs, 0)
    pltpu.sync_copy(node_v, rs_hbm.at[pl.ds(cid * NPAD + nbase, PT_NODES)])


@functools.partial(
    pl.kernel,
    mesh=_MESH,
    out_type=jax.ShapeDtypeStruct((N_EDGES_,), jnp.float32),
    scratch_types=[
        pltpu.VMEM_SHARED((NPAD,), jnp.float32),
        pltpu.VMEM_SHARED((NPAD,), jnp.float32),
        pltpu.VMEM((CHUNK_B,), jnp.int32),
        pltpu.VMEM((CHUNK_B,), jnp.int32),
        pltpu.VMEM((CHUNK_B,), jnp.float32),
        pltpu.VMEM((CHUNK_B,), jnp.int32),
        pltpu.VMEM((CHUNK_B,), jnp.int32),
        pltpu.VMEM((CHUNK_B,), jnp.float32),
        pltpu.VMEM((CHUNK_B,), jnp.float32),
        pltpu.VMEM((CHUNK_B,), jnp.float32),
        pltpu.VMEM((CHUNK_B,), jnp.float32),
        pltpu.VMEM((CHUNK_B,), jnp.float32),
        pltpu.VMEM((CHUNK_B,), jnp.float32),
        pltpu.VMEM((CHUNK_B,), jnp.float32),
        pltpu.VMEM((PT_NODES,), jnp.float32),
    ] + [pltpu.SemaphoreType.DMA] * 12,
)
def _apply(ew_hbm, ei_hbm, rs_hbm, out_hbm,
           rso_s, rsi_s,
           src0, dst0, ew0, src1, dst1, ew1,
           a0, b0, o0, a1, b1, o1, stage_v,
           ss0, sd0, se0, ss1, sd1, se1, sa0, sb0, so0, sa1, sb1, so1):
    cid = lax.axis_index("c")
    sid = lax.axis_index("s")
    wid = sid * 2 + cid
    nbase = sid * PT_NODES
    ns = pl.ds(nbase, PT_NODES)

    pltpu.sync_copy(rs_hbm.at[pl.ds(nbase, PT_NODES)], stage_v)
    pltpu.sync_copy(stage_v, rso_s.at[ns])
    pltpu.sync_copy(rs_hbm.at[pl.ds(NPAD + nbase, PT_NODES)], stage_v)
    pltpu.sync_copy(stage_v, rsi_s.at[ns])
    plsc.subcore_barrier()

    def _mul_store(av, bv, ev, ov):
        def _mul(i, carry2):
            for u in range(MUL_UNROLL):
                s = pl.ds((i * MUL_UNROLL + u) * 16, 16)
                ov[s] = av[s] * bv[s] * ev[s]
            return carry2
        lax.fori_loop(0, CHUNK_B // (16 * MUL_UNROLL), _mul, 0)

    def _chunk(k, carry):
        p0 = wid * PER_W + (2 * k) * CHUNK_B
        p1 = p0 + CHUNK_B
        cs0 = pltpu.async_copy(ei_hbm.at[pl.ds(p0, CHUNK_B)], src0, ss0)
        cd0 = pltpu.async_copy(ei_hbm.at[pl.ds(N_EDGES_ + p0, CHUNK_B)], dst0, sd0)
        ce0 = pltpu.async_copy(ew_hbm.at[pl.ds(p0, CHUNK_B)], ew0, se0)
        cs1 = pltpu.async_copy(ei_hbm.at[pl.ds(p1, CHUNK_B)], src1, ss1)
        cd1 = pltpu.async_copy(ei_hbm.at[pl.ds(N_EDGES_ + p1, CHUNK_B)], dst1, sd1)
        ce1 = pltpu.async_copy(ew_hbm.at[pl.ds(p1, CHUNK_B)], ew1, se1)
        cs0.wait()
        cd0.wait()
        ca0 = pltpu.async_copy(rso_s.at[src0], a0, sa0)
        cb0 = pltpu.async_copy(rsi_s.at[dst0], b0, sb0)
        ce0.wait()
        ca0.wait()
        cb0.wait()
        _mul_store(a0, b0, ew0, o0)
        co0 = pltpu.async_copy(o0, out_hbm.at[pl.ds(p0, CHUNK_B)], so0)
        cs1.wait()
        cd1.wait()
        ca1 = pltpu.async_copy(rso_s.at[src1], a1, sa1)
        cb1 = pltpu.async_copy(rsi_s.at[dst1], b1, sb1)
        ce1.wait()
        ca1.wait()
        cb1.wait()
        _mul_store(a1, b1, ew1, o1)
        co1 = pltpu.async_copy(o1, out_hbm.at[pl.ds(p1, CHUNK_B)], so1)
        co0.wait()
        co1.wait()
        return carry
    lax.fori_loop(0, CHUNKS_B // 2, _chunk, 0)


@jax.jit
def kernel(edge_weight, edge_index):
    ew = edge_weight.astype(jnp.float32)
    idx = edge_index.astype(jnp.int32).reshape(-1)
    rs = _norms(ew, idx)
    return _apply(ew, idx, rs)


# speedup vs baseline: 301.5765x; 1.0516x over previous
import functools

import jax
import jax.numpy as jnp
from jax import lax
from jax.experimental import pallas as pl
from jax.experimental.pallas import tpu as pltpu
from jax.experimental.pallas import tpu_sc as plsc

N_NODES_ = 100000
N_EDGES_ = 6400000

NPAD = 100096
PT_NODES = NPAD // 16

CHUNK_A = 10000
CHUNKS_A = (N_EDGES_ // 16) // CHUNK_A
CHUNK_B = 2000
PER_W = N_EDGES_ // 32
CHUNKS_B = PER_W // CHUNK_B
MUL_UNROLL = 5

_MESH = plsc.VectorSubcoreMesh(core_axis_name="c", subcore_axis_name="s")


def _rsqrt16(x):
    i = lax.bitcast_convert_type(x, jnp.int32)
    i = jnp.int32(0x5F3759DF) - lax.shift_right_logical(i, 1)
    y = lax.bitcast_convert_type(i, jnp.float32)
    for _ in range(3):
        y = y * (1.5 - 0.5 * x * y * y)
    return y


@functools.partial(
    pl.kernel,
    mesh=_MESH,
    out_type=jax.ShapeDtypeStruct((2 * NPAD,), jnp.float32),
    scratch_types=[
        pltpu.VMEM_SHARED((NPAD,), jnp.float32),
        pltpu.VMEM((CHUNK_A,), jnp.int32),
        pltpu.VMEM((CHUNK_A,), jnp.float32),
        pltpu.VMEM((CHUNK_A,), jnp.int32),
        pltpu.VMEM((CHUNK_A,), jnp.float32),
        pltpu.VMEM((PT_NODES,), jnp.float32),
        pltpu.SemaphoreType.DMA,
        pltpu.SemaphoreType.DMA,
        pltpu.SemaphoreType.DMA,
        pltpu.SemaphoreType.DMA,
    ],
)
def _norms(ew_hbm, ei_hbm, rs_hbm,
           acc, idx0, val0, idx1, val1, node_v, si0, sv0, si1, sv1):
    cid = lax.axis_index("c")
    sid = lax.axis_index("s")
    nbase = sid * PT_NODES

    def _zero(i, carry):
        node_v[pl.ds(i * 16, 16)] = jnp.zeros((16,), jnp.float32)
        return carry
    lax.fori_loop(0, PT_NODES // 16, _zero, 0)
    pltpu.sync_copy(node_v, acc.at[pl.ds(nbase, PT_NODES)])
    plsc.subcore_barrier()

    def _scat(k, carry):
        b0 = (sid * CHUNKS_A + 2 * k) * CHUNK_A
        b1 = b0 + CHUNK_A
        row = cid * N_EDGES_
        ci0 = pltpu.async_copy(ei_hbm.at[pl.ds(row + b0, CHUNK_A)], idx0, si0)
        cv0 = pltpu.async_copy(ew_hbm.at[pl.ds(b0, CHUNK_A)], val0, sv0)
        ci1 = pltpu.async_copy(ei_hbm.at[pl.ds(row + b1, CHUNK_A)], idx1, si1)
        cv1 = pltpu.async_copy(ew_hbm.at[pl.ds(b1, CHUNK_A)], val1, sv1)
        ci0.wait()
        cv0.wait()
        pltpu.sync_copy(val0, acc.at[idx0], add=True)
        ci1.wait()
        cv1.wait()
        pltpu.sync_copy(val1, acc.at[idx1], add=True)
        return carry
    lax.fori_loop(0, CHUNKS_A // 2, _scat, 0)
    plsc.subcore_barrier()

    pltpu.sync_copy(acc.at[pl.ds(nbase, PT_NODES)], node_v)

    def _rs(i, carry):
        s = pl.ds(i * 16, 16)
        node_v[s] = _rsqrt16(node_v[s])
        return carry
    lax.fori_loop(0, PT_NODES // 16, _rs, 0)
    pltpu.sync_copy(node_v, rs_hbm.at[pl.ds(cid * NPAD + nbase, PT_NODES)])


@functools.partial(
    pl.kernel,
    mesh=_MESH,
    out_type=jax.ShapeDtypeStruct((N_EDGES_,), jnp.float32),
    scratch_types=[
        pltpu.VMEM_SHARED((NPAD,), jnp.float32),
        pltpu.VMEM((NPAD,), jnp.float32),
        pltpu.VMEM((CHUNK_B,), jnp.int32),
        pltpu.VMEM((CHUNK_B,), jnp.int32),
        pltpu.VMEM((CHUNK_B,), jnp.float32),
        pltpu.VMEM((CHUNK_B,), jnp.int32),
        pltpu.VMEM((CHUNK_B,), jnp.int32),
        pltpu.VMEM((CHUNK_B,), jnp.float32),
        pltpu.VMEM((CHUNK_B,), jnp.float32),
        pltpu.VMEM((CHUNK_B,), jnp.float32),
        pltpu.VMEM((CHUNK_B,), jnp.float32),
        pltpu.VMEM((CHUNK_B,), jnp.float32),
    ] + [pltpu.SemaphoreType.DMA] * 10,
    compiler_params=pltpu.CompilerParams(needs_layout_passes=False),
)
def _apply(ew_hbm, ei_hbm, rs_hbm, out_hbm,
           rsi_s, tbl_v,
           src0, dst0, ew0, src1, dst1, ew1,
           b0, o0, b1, o1,
           ss0, sd0, se0, ss1, sd1, se1, sb0, sb1, so0, so1):
    cid = lax.axis_index("c")
    sid = lax.axis_index("s")
    wid = sid * 2 + cid
    nbase = sid * PT_NODES
    ns = pl.ds(nbase, PT_NODES)

    pltpu.sync_copy(rs_hbm.at[pl.ds(NPAD + nbase, PT_NODES)],
                    tbl_v.at[pl.ds(0, PT_NODES)])
    pltpu.sync_copy(tbl_v.at[pl.ds(0, PT_NODES)], rsi_s.at[ns])
    pltpu.sync_copy(rs_hbm.at[pl.ds(0, NPAD)], tbl_v)
    plsc.subcore_barrier()

    def _mul_store(sv, bv, ev, ov):
        def _mul(i, carry2):
            for u in range(MUL_UNROLL):
                s = pl.ds((i * MUL_UNROLL + u) * 16, 16)
                a = plsc.load_gather(tbl_v, [sv[s]])
                ov[s] = a * bv[s] * ev[s]
            return carry2
        lax.fori_loop(0, CHUNK_B // (16 * MUL_UNROLL), _mul, 0)

    def _chunk(k, carry):
        p0 = wid * PER_W + (2 * k) * CHUNK_B
        p1 = p0 + CHUNK_B
        cs0 = pltpu.async_copy(ei_hbm.at[pl.ds(p0, CHUNK_B)], src0, ss0)
        cd0 = pltpu.async_copy(ei_hbm.at[pl.ds(N_EDGES_ + p0, CHUNK_B)], dst0, sd0)
        ce0 = pltpu.async_copy(ew_hbm.at[pl.ds(p0, CHUNK_B)], ew0, se0)
        cs1 = pltpu.async_copy(ei_hbm.at[pl.ds(p1, CHUNK_B)], src1, ss1)
        cd1 = pltpu.async_copy(ei_hbm.at[pl.ds(N_EDGES_ + p1, CHUNK_B)], dst1, sd1)
        ce1 = pltpu.async_copy(ew_hbm.at[pl.ds(p1, CHUNK_B)], ew1, se1)
        cd0.wait()
        cb0 = pltpu.async_copy(rsi_s.at[dst0], b0, sb0)
        cs0.wait()
        ce0.wait()
        cb0.wait()
        _mul_store(src0, b0, ew0, o0)
        co0 = pltpu.async_copy(o0, out_hbm.at[pl.ds(p0, CHUNK_B)], so0)
        cd1.wait()
        cb1 = pltpu.async_copy(rsi_s.at[dst1], b1, sb1)
        cs1.wait()
        ce1.wait()
        cb1.wait()
        _mul_store(src1, b1, ew1, o1)
        co1 = pltpu.async_copy(o1, out_hbm.at[pl.ds(p1, CHUNK_B)], so1)
        co0.wait()
        co1.wait()
        return carry
    lax.fori_loop(0, CHUNKS_B // 2, _chunk, 0)


@jax.jit
def kernel(edge_weight, edge_index):
    ew = edge_weight.astype(jnp.float32)
    idx = edge_index.astype(jnp.int32).reshape(-1)
    rs = _norms(ew, idx)
    return _apply(ew, idx, rs)


# speedup vs baseline: 341.6053x; 1.1327x over previous
import functools

import jax
import jax.numpy as jnp
from jax import lax
from jax.experimental import pallas as pl
from jax.experimental.pallas import tpu as pltpu
from jax.experimental.pallas import tpu_sc as plsc

N_NODES_ = 100000
N_EDGES_ = 6400000

NPAD = 100096
PT_NODES = NPAD // 16

CHUNK_A = 10000
CHUNKS_A = (N_EDGES_ // 16) // CHUNK_A
CHUNK_B = 2000
PER_W = N_EDGES_ // 32
CHUNKS_B = PER_W // CHUNK_B
MUL_UNROLL = 5

_MESH = plsc.VectorSubcoreMesh(core_axis_name="c", subcore_axis_name="s")


def _rsqrt16(x):
    i = lax.bitcast_convert_type(x, jnp.int32)
    i = jnp.int32(0x5F3759DF) - lax.shift_right_logical(i, 1)
    y = lax.bitcast_convert_type(i, jnp.float32)
    for _ in range(3):
        y = y * (1.5 - 0.5 * x * y * y)
    return y


@functools.partial(
    pl.kernel,
    mesh=_MESH,
    out_type=jax.ShapeDtypeStruct((2 * NPAD,), jnp.float32),
    scratch_types=[
        pltpu.VMEM_SHARED((NPAD,), jnp.float32),
        pltpu.VMEM((CHUNK_A,), jnp.int32),
        pltpu.VMEM((CHUNK_A,), jnp.float32),
        pltpu.VMEM((CHUNK_A,), jnp.int32),
        pltpu.VMEM((CHUNK_A,), jnp.float32),
        pltpu.VMEM((PT_NODES,), jnp.float32),
        pltpu.SemaphoreType.DMA,
        pltpu.SemaphoreType.DMA,
        pltpu.SemaphoreType.DMA,
        pltpu.SemaphoreType.DMA,
    ],
)
def _norms(ew_hbm, ei_hbm, rs_hbm,
           acc, idx0, val0, idx1, val1, node_v, si0, sv0, si1, sv1):
    cid = lax.axis_index("c")
    sid = lax.axis_index("s")
    nbase = sid * PT_NODES

    def _zero(i, carry):
        node_v[pl.ds(i * 16, 16)] = jnp.zeros((16,), jnp.float32)
        return carry
    lax.fori_loop(0, PT_NODES // 16, _zero, 0)
    pltpu.sync_copy(node_v, acc.at[pl.ds(nbase, PT_NODES)])
    plsc.subcore_barrier()

    def _scat(k, carry):
        b0 = (sid * CHUNKS_A + 2 * k) * CHUNK_A
        b1 = b0 + CHUNK_A
        row = cid * N_EDGES_
        ci0 = pltpu.async_copy(ei_hbm.at[pl.ds(row + b0, CHUNK_A)], idx0, si0)
        cv0 = pltpu.async_copy(ew_hbm.at[pl.ds(b0, CHUNK_A)], val0, sv0)
        ci1 = pltpu.async_copy(ei_hbm.at[pl.ds(row + b1, CHUNK_A)], idx1, si1)
        cv1 = pltpu.async_copy(ew_hbm.at[pl.ds(b1, CHUNK_A)], val1, sv1)
        ci0.wait()
        cv0.wait()
        pltpu.sync_copy(val0, acc.at[idx0], add=True)
        ci1.wait()
        cv1.wait()
        pltpu.sync_copy(val1, acc.at[idx1], add=True)
        return carry
    lax.fori_loop(0, CHUNKS_A // 2, _scat, 0)
    plsc.subcore_barrier()

    pltpu.sync_copy(acc.at[pl.ds(nbase, PT_NODES)], node_v)

    def _rs(i, carry):
        s = pl.ds(i * 16, 16)
        node_v[s] = _rsqrt16(node_v[s])
        return carry
    lax.fori_loop(0, PT_NODES // 16, _rs, 0)
    pltpu.sync_copy(node_v, rs_hbm.at[pl.ds(cid * NPAD + nbase, PT_NODES)])


_PACK_PIECES = ((0, 2000), (2000, 2000), (4000, 2000), (6000, 256))


@functools.partial(
    pl.kernel,
    mesh=_MESH,
    out_type=jax.ShapeDtypeStruct((N_EDGES_,), jnp.float32),
    scratch_types=[
        pltpu.VMEM_SHARED((NPAD,), jnp.int32),
        pltpu.VMEM((NPAD,), jnp.int32),
        pltpu.VMEM((CHUNK_B,), jnp.int32),
        pltpu.VMEM((CHUNK_B,), jnp.int32),
        pltpu.VMEM((CHUNK_B,), jnp.float32),
        pltpu.VMEM((CHUNK_B,), jnp.int32),
        pltpu.VMEM((CHUNK_B,), jnp.int32),
        pltpu.VMEM((CHUNK_B,), jnp.float32),
        pltpu.VMEM((CHUNK_B,), jnp.int32),
    ] + [pltpu.SemaphoreType.DMA] * 8,
    compiler_params=pltpu.CompilerParams(needs_layout_passes=False),
)
def _apply(ew_hbm, ei_hbm, rs_hbm, out_hbm,
           pack_s, tbl_v,
           src0, dst0, ew0, src1, dst1, ew1, pk,
           ss0, sd0, se0, ss1, sd1, se1, so0, so1):
    cid = lax.axis_index("c")
    sid = lax.axis_index("s")
    wid = sid * 2 + cid
    nbase = sid * PT_NODES

    for poff, psz in _PACK_PIECES:
        ca = pltpu.async_copy(rs_hbm.at[pl.ds(nbase + poff, psz)],
                              ew0.at[pl.ds(0, psz)], se0)
        cb = pltpu.async_copy(rs_hbm.at[pl.ds(NPAD + nbase + poff, psz)],
                              ew1.at[pl.ds(0, psz)], se1)
        ca.wait()
        cb.wait()

        def _pk(i, carry):
            s = pl.ds(i * 16, 16)
            ob = lax.bitcast_convert_type(ew0[s], jnp.int32) + jnp.int32(0x8000)
            ib = lax.bitcast_convert_type(ew1[s], jnp.int32) + jnp.int32(0x8000)
            pk[s] = jnp.bitwise_or(
                jnp.bitwise_and(ib, jnp.int32(-65536)),
                lax.shift_right_logical(ob, 16),
            )
            return carry
        lax.fori_loop(0, psz // 16, _pk, 0)
        pltpu.sync_copy(pk.at[pl.ds(0, psz)],
                        pack_s.at[pl.ds(nbase + poff, psz)])
    plsc.subcore_barrier()
    pltpu.sync_copy(pack_s, tbl_v)

    def _mul_store(sv, dv, ev):
        def _mul(i, carry2):
            for u in range(MUL_UNROLL):
                s = pl.ds((i * MUL_UNROLL + u) * 16, 16)
                wa = plsc.load_gather(tbl_v, [sv[s]])
                wb = plsc.load_gather(tbl_v, [dv[s]])
                a = lax.bitcast_convert_type(
                    lax.shift_left(wa, 16), jnp.float32)
                b = lax.bitcast_convert_type(
                    jnp.bitwise_and(wb, jnp.int32(-65536)), jnp.float32)
                ev[s] = a * b * ev[s]
            return carry2
        lax.fori_loop(0, CHUNK_B // (16 * MUL_UNROLL), _mul, 0)

    def _chunk(k, carry):
        p0 = wid * PER_W + (2 * k) * CHUNK_B
        p1 = p0 + CHUNK_B
        cs0 = pltpu.async_copy(ei_hbm.at[pl.ds(p0, CHUNK_B)], src0, ss0)
        cd0 = pltpu.async_copy(ei_hbm.at[pl.ds(N_EDGES_ + p0, CHUNK_B)], dst0, sd0)
        ce0 = pltpu.async_copy(ew_hbm.at[pl.ds(p0, CHUNK_B)], ew0, se0)
        cs1 = pltpu.async_copy(ei_hbm.at[pl.ds(p1, CHUNK_B)], src1, ss1)
        cd1 = pltpu.async_copy(ei_hbm.at[pl.ds(N_EDGES_ + p1, CHUNK_B)], dst1, sd1)
        ce1 = pltpu.async_copy(ew_hbm.at[pl.ds(p1, CHUNK_B)], ew1, se1)
        cs0.wait()
        cd0.wait()
        ce0.wait()
        _mul_store(src0, dst0, ew0)
        co0 = pltpu.async_copy(ew0, out_hbm.at[pl.ds(p0, CHUNK_B)], so0)
        cs1.wait()
        cd1.wait()
        ce1.wait()
        _mul_store(src1, dst1, ew1)
        co1 = pltpu.async_copy(ew1, out_hbm.at[pl.ds(p1, CHUNK_B)], so1)
        co0.wait()
        co1.wait()
        return carry
    lax.fori_loop(0, CHUNKS_B // 2, _chunk, 0)


@jax.jit
def kernel(edge_weight, edge_index):
    ew = edge_weight.astype(jnp.float32)
    idx = edge_index.astype(jnp.int32).reshape(-1)
    rs = _norms(ew, idx)
    return _apply(ew, idx, rs)
